# baseline (speedup 1.0000x reference)
import functools

import jax
import jax.numpy as jnp
from jax import lax
from jax.experimental import pallas as pl
from jax.experimental.pallas import tpu as pltpu
from jax.experimental.pallas import tpu_sc as plsc

E_BLOCK = 8000
N_BLOCK = 2000
TD = 48
GCH = 1000
N_WORKERS = 32


def _silu(v):
    return v * jax.nn.sigmoid(v)


def _make_gather(E):
    per_w = E // N_WORKERS
    nch = per_w // GCH
    mesh = plsc.VectorSubcoreMesh(core_axis_name="c", subcore_axis_name="s")

    @functools.partial(
        pl.kernel,
        out_type=jax.ShapeDtypeStruct((E, 128), jnp.float32),
        mesh=mesh,
        scratch_types=[
            pltpu.VMEM((GCH,), jnp.int32),
            pltpu.VMEM((GCH,), jnp.int32),
            pltpu.VMEM((GCH, TD), jnp.float32),
            pltpu.VMEM((GCH, TD), jnp.float32),
            pltpu.VMEM((GCH, 16), jnp.float32),
            pltpu.SemaphoreType.DMA,
            pltpu.SemaphoreType.DMA,
            pltpu.SemaphoreType.DMA,
        ],
        compiler_params=pltpu.CompilerParams(use_tc_tiling_on_sc=False),
    )
    def gather_k(t_hbm, row_hbm, col_hbm, ea_hbm, out_hbm,
                 idxr_v, idxc_v, bufr_v, bufc_v, bufe_v, semr, semc, seme):
        wid = lax.axis_index("s") * 2 + lax.axis_index("c")
        base = wid * per_w

        def body(i, carry):
            off = base + i * GCH
            pltpu.sync_copy(row_hbm.at[pl.ds(off, GCH)], idxr_v)
            pltpu.sync_copy(col_hbm.at[pl.ds(off, GCH)], idxc_v)
            ce = pltpu.async_copy(ea_hbm.at[pl.ds(off, GCH)], bufe_v, seme)
            cr = pltpu.async_copy(t_hbm.at[idxr_v], bufr_v, semr)
            cc = pltpu.async_copy(t_hbm.at[idxc_v], bufc_v, semc)
            cr.wait()
            pltpu.sync_copy(bufr_v, out_hbm.at[pl.ds(off, GCH), pl.ds(0, TD)])
            cc.wait()
            pltpu.sync_copy(bufc_v, out_hbm.at[pl.ds(off, GCH), pl.ds(TD, TD)])
            ce.wait()
            pltpu.sync_copy(bufe_v, out_hbm.at[pl.ds(off, GCH), pl.ds(2 * TD, 16)])
            return carry

        lax.fori_loop(0, nch, body, 0)

    return gather_k


SCH = 1000


def _make_scatter(E, N):
    per_tile = E // 16
    nch1 = per_tile // SCH
    per_tile2 = E // 32
    nch2 = per_tile2 // SCH
    stripe = N // 16
    mesh = plsc.VectorSubcoreMesh(core_axis_name="c", subcore_axis_name="s")

    @functools.partial(
        pl.kernel,
        out_type=[
            jax.ShapeDtypeStruct((2, N, 16), jnp.float32),
            jax.ShapeDtypeStruct((2, N, 16), jnp.float32),
        ],
        mesh=mesh,
        scratch_types=[
            pltpu.VMEM((SCH,), jnp.int32),
            pltpu.VMEM((SCH, 16), jnp.float32),
            pltpu.VMEM_SHARED((N, 16), jnp.float32),
        ],
        compiler_params=pltpu.CompilerParams(use_tc_tiling_on_sc=False),
    )
    def scatter_k(msgs_hbm, col_hbm, zs_hbm, outs_hbm, outv_hbm,
                  idx_v, buf_v, acc_sh):
        cid = lax.axis_index("c")
        tid = lax.axis_index("s")
        zslice = pl.ds(tid * stripe, stripe)

        def run_phase(base, nch, lane0, out_hbm):
            pltpu.sync_copy(zs_hbm.at[zslice], acc_sh.at[zslice])
            plsc.subcore_barrier()

            def body(i, carry):
                off = base + i * SCH
                pltpu.sync_copy(col_hbm.at[pl.ds(off, SCH)], idx_v)
                pltpu.sync_copy(
                    msgs_hbm.at[pl.ds(off, SCH), pl.ds(lane0, 16)], buf_v)
                pltpu.sync_copy(buf_v, acc_sh.at[idx_v], add=True)
                return carry

            lax.fori_loop(0, nch, body, 0)
            plsc.subcore_barrier()
            pltpu.sync_copy(acc_sh.at[zslice], out_hbm.at[cid, zslice])
            plsc.subcore_barrier()

        run_phase(tid * per_tile, nch1, cid * 16, outs_hbm)
        run_phase(cid * (E // 2) + tid * per_tile2, nch2, 32, outv_hbm)

    return scatter_k


def _edge_body(in_ref, xmask_ref,
               w0_ref, b0_ref, w1_ref, b1_ref, w2_ref, b2_ref,
               vw0_ref, vb0_ref, vw1_ref, vb1_ref, vw2_ref, vb2_ref,
               sw0_ref, sb0_ref, sw1_ref, sb1_ref, sw2_ref, sb2_ref,
               out_ref):
    mm = lambda a, w: a @ w
    g = in_ref[...]
    xm = xmask_ref[...]
    n2 = jnp.sum(g * g * xm, axis=1, keepdims=True)
    inv = 1.0 / jnp.maximum(jnp.sqrt(n2), 1e-12)
    scale = xm * inv + (1.0 - xm)
    pre = mm(g * scale, w0_ref[...]) + b0_ref[...]
    l1 = _silu(pre)
    l2 = _silu(mm(l1, w1_ref[...]) + b1_ref[...])
    msg = mm(l2, w2_ref[...]) + b2_ref[...]
    v = _silu(mm(msg, vw0_ref[...]) + vb0_ref[...])
    v = _silu(mm(v, vw1_ref[...]) + vb1_ref[...])
    vm = mm(v, vw2_ref[...]) + vb2_ref[...]
    s = _silu(mm(msg, sw0_ref[...]) + sb0_ref[...])
    s = _silu(mm(s, sw1_ref[...]) + sb1_ref[...])
    sm = mm(s, sw2_ref[...]) + sb2_ref[...]
    pad = jnp.zeros((sm.shape[0], 13), sm.dtype)
    out_ref[...] = jnp.concatenate([sm, vm, pad], axis=1)


def _node_body(agg_ref, h_ref, vs_ref, x_ref,
               w0_ref, b0_ref, w1_ref, b1_ref, w2_ref, b2_ref,
               vec_ref, na_ref):
    a = _silu(agg_ref[...] @ w0_ref[...] + b0_ref[...])
    a = _silu(a @ w1_ref[...] + b1_ref[...])
    na_ref[...] = (a @ w2_ref[...] + b2_ref[...]) + h_ref[...]
    vec_ref[...] = x_ref[...] + vs_ref[...]


def _full(shape):
    return pl.BlockSpec(shape, lambda i: (0,) * len(shape))


def kernel(edge_index, edge_attr, x, h,
           msg_W0, msg_b0, msg_W1, msg_b1, msg_W2, msg_b2,
           vec_W0, vec_b0, vec_W1, vec_b1, vec_W2, vec_b2,
           sca_W0, sca_b0, sca_W1, sca_b1, sca_W2, sca_b2,
           nod_W0, nod_b0, nod_W1, nod_b1, nod_W2, nod_b2):
    E = edge_index.shape[1]
    N = x.shape[0]
    row = edge_index[0]
    col = edge_index[1]

    T = jnp.concatenate(
        [x, jnp.zeros((N, 5), jnp.float32), h, jnp.zeros((N, 8), jnp.float32)],
        axis=1)

    packed = _make_gather(E)(T, row, col, edge_attr)

    z = jnp.zeros((5, 64), jnp.float32)
    z8 = jnp.zeros((8, 64), jnp.float32)
    w0p = jnp.concatenate([
        msg_W0[0:3], z, msg_W0[6:38], z8,
        msg_W0[3:6], z, msg_W0[38:70], z8,
        msg_W0[70:86], jnp.zeros((16, 64), jnp.float32),
    ], axis=0)
    xmask = jnp.zeros((1, 128), jnp.float32)
    xmask = xmask.at[0, 0:3].set(1.0).at[0, 48:51].set(1.0)
    b2 = lambda b: b.reshape(1, -1)
    bfw = lambda w: w

    grid_e = E // E_BLOCK
    msgs = pl.pallas_call(
        _edge_body,
        grid=(grid_e,),
        in_specs=[
            pl.BlockSpec((E_BLOCK, 128), lambda i: (i, 0)),
            _full((1, 128)),
            _full((128, 64)), _full((1, 64)),
            _full(msg_W1.shape), _full((1, 64)), _full(msg_W2.shape), _full((1, 64)),
            _full(vec_W0.shape), _full((1, 64)), _full(vec_W1.shape), _full((1, 64)),
            _full(vec_W2.shape), _full((1, 3)),
            _full(sca_W0.shape), _full((1, 64)), _full(sca_W1.shape), _full((1, 64)),
            _full(sca_W2.shape), _full((1, 32)),
        ],
        out_specs=pl.BlockSpec((E_BLOCK, 48), lambda i: (i, 0)),
        out_shape=jax.ShapeDtypeStruct((E, 48), jnp.float32),
        compiler_params=pltpu.CompilerParams(
            dimension_semantics=("arbitrary",),
        ),
    )(packed, xmask,
      bfw(w0p), b2(msg_b0), bfw(msg_W1), b2(msg_b1), bfw(msg_W2), b2(msg_b2),
      bfw(vec_W0), b2(vec_b0), bfw(vec_W1), b2(vec_b1), bfw(vec_W2), b2(vec_b2),
      bfw(sca_W0), b2(sca_b0), bfw(sca_W1), b2(sca_b1), bfw(sca_W2), b2(sca_b2))

    zs = jnp.zeros((N, 16), jnp.float32)
    parts_s, parts_v = _make_scatter(E, N)(msgs, col, zs)
    agg = jnp.concatenate([parts_s[0], parts_s[1]], axis=1)
    vecsum = (parts_v[0] + parts_v[1])[:, 0:3]

    grid_n = N // N_BLOCK
    vector, node_attr = pl.pallas_call(
        _node_body,
        grid=(grid_n,),
        in_specs=[
            pl.BlockSpec((N_BLOCK, 32), lambda i: (i, 0)),
            pl.BlockSpec((N_BLOCK, 32), lambda i: (i, 0)),
            pl.BlockSpec((N_BLOCK, 3), lambda i: (i, 0)),
            pl.BlockSpec((N_BLOCK, 3), lambda i: (i, 0)),
            _full(nod_W0.shape), _full((1, 64)), _full(nod_W1.shape), _full((1, 64)),
            _full(nod_W2.shape), _full((1, 32)),
        ],
        out_specs=[
            pl.BlockSpec((N_BLOCK, 3), lambda i: (i, 0)),
            pl.BlockSpec((N_BLOCK, 32), lambda i: (i, 0)),
        ],
        out_shape=[
            jax.ShapeDtypeStruct((N, 3), jnp.float32),
            jax.ShapeDtypeStruct((N, 32), jnp.float32),
        ],
        compiler_params=pltpu.CompilerParams(
            dimension_semantics=("arbitrary",),
        ),
    )(agg, h, vecsum, x,
      nod_W0, b2(nod_b0), nod_W1, b2(nod_b1), nod_W2, b2(nod_b2))

    return vector, node_attr


# speedup vs baseline: 1.0813x; 1.0813x over previous
import functools

import jax
import jax.numpy as jnp
from jax import lax
from jax.experimental import pallas as pl
from jax.experimental.pallas import tpu as pltpu
from jax.experimental.pallas import tpu_sc as plsc

E_BLOCK = 8000
N_BLOCK = 2000
TD = 48
GCH = 1000
N_WORKERS = 32


def _silu(v):
    return v * (0.5 * jnp.tanh(0.5 * v) + 0.5)


def _make_gather(E):
    per_w = E // N_WORKERS
    nch = per_w // GCH
    mesh = plsc.VectorSubcoreMesh(core_axis_name="c", subcore_axis_name="s")

    @functools.partial(
        pl.kernel,
        out_type=jax.ShapeDtypeStruct((E, 128), jnp.float32),
        mesh=mesh,
        scratch_types=[
            pltpu.VMEM((GCH,), jnp.int32),
            pltpu.VMEM((GCH,), jnp.int32),
            pltpu.VMEM((GCH, TD), jnp.float32),
            pltpu.VMEM((GCH, TD), jnp.float32),
            pltpu.VMEM((GCH, 16), jnp.float32),
            pltpu.SemaphoreType.DMA,
            pltpu.SemaphoreType.DMA,
            pltpu.SemaphoreType.DMA,
        ],
        compiler_params=pltpu.CompilerParams(use_tc_tiling_on_sc=False),
    )
    def gather_k(t_hbm, row_hbm, col_hbm, ea_hbm, out_hbm,
                 idxr_v, idxc_v, bufr_v, bufc_v, bufe_v, semr, semc, seme):
        wid = lax.axis_index("s") * 2 + lax.axis_index("c")
        base = wid * per_w

        def body(i, carry):
            off = base + i * GCH
            pltpu.sync_copy(row_hbm.at[pl.ds(off, GCH)], idxr_v)
            pltpu.sync_copy(col_hbm.at[pl.ds(off, GCH)], idxc_v)
            ce = pltpu.async_copy(ea_hbm.at[pl.ds(off, GCH)], bufe_v, seme)
            cr = pltpu.async_copy(t_hbm.at[idxr_v], bufr_v, semr)
            cc = pltpu.async_copy(t_hbm.at[idxc_v], bufc_v, semc)
            cr.wait()
            pltpu.sync_copy(bufr_v, out_hbm.at[pl.ds(off, GCH), pl.ds(0, TD)])
            cc.wait()
            pltpu.sync_copy(bufc_v, out_hbm.at[pl.ds(off, GCH), pl.ds(TD, TD)])
            ce.wait()
            pltpu.sync_copy(bufe_v, out_hbm.at[pl.ds(off, GCH), pl.ds(2 * TD, 16)])
            return carry

        lax.fori_loop(0, nch, body, 0)

    return gather_k


SCH = 1000


def _make_scatter(E, N):
    per_tile = E // 16
    nch1 = per_tile // SCH
    per_tile2 = E // 32
    nch2 = per_tile2 // SCH
    stripe = N // 16
    mesh = plsc.VectorSubcoreMesh(core_axis_name="c", subcore_axis_name="s")

    @functools.partial(
        pl.kernel,
        out_type=[
            jax.ShapeDtypeStruct((2, N, 16), jnp.float32),
            jax.ShapeDtypeStruct((2, N, 16), jnp.float32),
        ],
        mesh=mesh,
        scratch_types=[
            pltpu.VMEM((SCH,), jnp.int32),
            pltpu.VMEM((SCH, 16), jnp.float32),
            pltpu.VMEM_SHARED((N, 16), jnp.float32),
        ],
        compiler_params=pltpu.CompilerParams(use_tc_tiling_on_sc=False),
    )
    def scatter_k(msgs_hbm, col_hbm, zs_hbm, outs_hbm, outv_hbm,
                  idx_v, buf_v, acc_sh):
        cid = lax.axis_index("c")
        tid = lax.axis_index("s")
        zslice = pl.ds(tid * stripe, stripe)

        def run_phase(base, nch, lane0, out_hbm):
            pltpu.sync_copy(zs_hbm.at[zslice], acc_sh.at[zslice])
            plsc.subcore_barrier()

            def body(i, carry):
                off = base + i * SCH
                pltpu.sync_copy(col_hbm.at[pl.ds(off, SCH)], idx_v)
                pltpu.sync_copy(
                    msgs_hbm.at[pl.ds(off, SCH), pl.ds(lane0, 16)], buf_v)
                pltpu.sync_copy(buf_v, acc_sh.at[idx_v], add=True)
                return carry

            lax.fori_loop(0, nch, body, 0)
            plsc.subcore_barrier()
            pltpu.sync_copy(acc_sh.at[zslice], out_hbm.at[cid, zslice])
            plsc.subcore_barrier()

        run_phase(tid * per_tile, nch1, cid * 16, outs_hbm)
        run_phase(cid * (E // 2) + tid * per_tile2, nch2, 32, outv_hbm)

    return scatter_k


def _edge_body(in_ref, xmask_ref,
               w0_ref, b0_ref, w1_ref, b1_ref, w2_ref, b2_ref,
               vw0_ref, vb0_ref, vw1_ref, vb1_ref, vw2_ref, vb2_ref,
               sw0_ref, sb0_ref, sw1_ref, sb1_ref, sw2_ref, sb2_ref,
               out_ref):
    mm = lambda a, w: a @ w
    g = in_ref[...]
    xm = xmask_ref[...]
    n2 = jnp.sum(g * g * xm, axis=1, keepdims=True)
    inv = lax.rsqrt(jnp.maximum(n2, 1e-24))
    scale = xm * inv + (1.0 - xm)
    pre = mm(g * scale, w0_ref[...]) + b0_ref[...]
    l1 = _silu(pre)
    l2 = _silu(mm(l1, w1_ref[...]) + b1_ref[...])
    msg = mm(l2, w2_ref[...]) + b2_ref[...]
    v = _silu(mm(msg, vw0_ref[...]) + vb0_ref[...])
    v = _silu(mm(v, vw1_ref[...]) + vb1_ref[...])
    vm = mm(v, vw2_ref[...]) + vb2_ref[...]
    s = _silu(mm(msg, sw0_ref[...]) + sb0_ref[...])
    s = _silu(mm(s, sw1_ref[...]) + sb1_ref[...])
    sm = mm(s, sw2_ref[...]) + sb2_ref[...]
    pad = jnp.zeros((sm.shape[0], 13), sm.dtype)
    out_ref[...] = jnp.concatenate([sm, vm, pad], axis=1)


def _node_body(agg_ref, h_ref, vs_ref, x_ref,
               w0_ref, b0_ref, w1_ref, b1_ref, w2_ref, b2_ref,
               vec_ref, na_ref):
    a = _silu(agg_ref[...] @ w0_ref[...] + b0_ref[...])
    a = _silu(a @ w1_ref[...] + b1_ref[...])
    na_ref[...] = (a @ w2_ref[...] + b2_ref[...]) + h_ref[...]
    vec_ref[...] = x_ref[...] + vs_ref[...]


def _full(shape):
    return pl.BlockSpec(shape, lambda i: (0,) * len(shape))


def kernel(edge_index, edge_attr, x, h,
           msg_W0, msg_b0, msg_W1, msg_b1, msg_W2, msg_b2,
           vec_W0, vec_b0, vec_W1, vec_b1, vec_W2, vec_b2,
           sca_W0, sca_b0, sca_W1, sca_b1, sca_W2, sca_b2,
           nod_W0, nod_b0, nod_W1, nod_b1, nod_W2, nod_b2):
    E = edge_index.shape[1]
    N = x.shape[0]
    row = edge_index[0]
    col = edge_index[1]

    T = jnp.concatenate(
        [x, jnp.zeros((N, 5), jnp.float32), h, jnp.zeros((N, 8), jnp.float32)],
        axis=1)

    packed = _make_gather(E)(T, row, col, edge_attr)

    z = jnp.zeros((5, 64), jnp.float32)
    z8 = jnp.zeros((8, 64), jnp.float32)
    w0p = jnp.concatenate([
        msg_W0[0:3], z, msg_W0[6:38], z8,
        msg_W0[3:6], z, msg_W0[38:70], z8,
        msg_W0[70:86], jnp.zeros((16, 64), jnp.float32),
    ], axis=0)
    xmask = jnp.zeros((1, 128), jnp.float32)
    xmask = xmask.at[0, 0:3].set(1.0).at[0, 48:51].set(1.0)
    b2 = lambda b: b.reshape(1, -1)
    bfw = lambda w: w

    grid_e = E // E_BLOCK
    msgs = pl.pallas_call(
        _edge_body,
        grid=(grid_e,),
        in_specs=[
            pl.BlockSpec((E_BLOCK, 128), lambda i: (i, 0)),
            _full((1, 128)),
            _full((128, 64)), _full((1, 64)),
            _full(msg_W1.shape), _full((1, 64)), _full(msg_W2.shape), _full((1, 64)),
            _full(vec_W0.shape), _full((1, 64)), _full(vec_W1.shape), _full((1, 64)),
            _full(vec_W2.shape), _full((1, 3)),
            _full(sca_W0.shape), _full((1, 64)), _full(sca_W1.shape), _full((1, 64)),
            _full(sca_W2.shape), _full((1, 32)),
        ],
        out_specs=pl.BlockSpec((E_BLOCK, 48), lambda i: (i, 0)),
        out_shape=jax.ShapeDtypeStruct((E, 48), jnp.float32),
        compiler_params=pltpu.CompilerParams(
            dimension_semantics=("arbitrary",),
        ),
    )(packed, xmask,
      bfw(w0p), b2(msg_b0), bfw(msg_W1), b2(msg_b1), bfw(msg_W2), b2(msg_b2),
      bfw(vec_W0), b2(vec_b0), bfw(vec_W1), b2(vec_b1), bfw(vec_W2), b2(vec_b2),
      bfw(sca_W0), b2(sca_b0), bfw(sca_W1), b2(sca_b1), bfw(sca_W2), b2(sca_b2))

    zs = jnp.zeros((N, 16), jnp.float32)
    parts_s, parts_v = _make_scatter(E, N)(msgs, col, zs)
    agg = jnp.concatenate([parts_s[0], parts_s[1]], axis=1)
    vecsum = (parts_v[0] + parts_v[1])[:, 0:3]

    grid_n = N // N_BLOCK
    vector, node_attr = pl.pallas_call(
        _node_body,
        grid=(grid_n,),
        in_specs=[
            pl.BlockSpec((N_BLOCK, 32), lambda i: (i, 0)),
            pl.BlockSpec((N_BLOCK, 32), lambda i: (i, 0)),
            pl.BlockSpec((N_BLOCK, 3), lambda i: (i, 0)),
            pl.BlockSpec((N_BLOCK, 3), lambda i: (i, 0)),
            _full(nod_W0.shape), _full((1, 64)), _full(nod_W1.shape), _full((1, 64)),
            _full(nod_W2.shape), _full((1, 32)),
        ],
        out_specs=[
            pl.BlockSpec((N_BLOCK, 3), lambda i: (i, 0)),
            pl.BlockSpec((N_BLOCK, 32), lambda i: (i, 0)),
        ],
        out_shape=[
            jax.ShapeDtypeStruct((N, 3), jnp.float32),
            jax.ShapeDtypeStruct((N, 32), jnp.float32),
        ],
        compiler_params=pltpu.CompilerParams(
            dimension_semantics=("arbitrary",),
        ),
    )(agg, h, vecsum, x,
      nod_W0, b2(nod_b0), nod_W1, b2(nod_b1), nod_W2, b2(nod_b2))

    return vector, node_attr
